# baseline (speedup 1.0000x reference)
import functools

import jax
import jax.numpy as jnp
from jax import lax
from jax.experimental import pallas as pl
from jax.experimental.pallas import tpu as pltpu
from jax.experimental.pallas import tpu_sc as plsc

_G = 256


def _make_gather(n_groups, V, D, K, NBUF):
    info = plsc.get_sparse_core_info()
    NC, NS = info.num_cores, info.num_subcores
    NW = NC * NS
    assert n_groups % NW == 0
    g_per_w = n_groups // NW
    assert g_per_w % (K * NBUF) == 0
    n_outer = g_per_w // (K * NBUF)
    mesh = plsc.VectorSubcoreMesh(core_axis_name="c", subcore_axis_name="s")

    sem_types = [pltpu.SemaphoreType.DMA] * (3 * NBUF)

    @functools.partial(
        pl.kernel,
        mesh=mesh,
        out_type=jax.ShapeDtypeStruct((NW, g_per_w, _G, D), jnp.float32),
        scratch_types=[
            pltpu.VMEM((NBUF, K, _G), jnp.int32),
            pltpu.VMEM((NBUF, K, _G, D), jnp.float32),
        ]
        + sem_types,
        compiler_params=pltpu.CompilerParams(use_tc_tiling_on_sc=False),
    )
    def k(tok_hbm, table_hbm, out_hbm, idx_v, rows_v, *sems):
        isems = sems[:NBUF]
        gsems = sems[NBUF : 2 * NBUF]
        ssems = sems[2 * NBUF :]
        wid = lax.axis_index("s") * NC + lax.axis_index("c")
        tok_w = tok_hbm.at[wid]
        out_w = out_hbm.at[wid]

        def outer_body(ci, carry):
            c0 = ci * NBUF
            iwaits = []
            for b in range(NBUF):
                iwaits.append(
                    pltpu.async_copy(
                        tok_w.at[pl.ds((c0 + b) * K, K)], idx_v.at[b], isems[b]
                    )
                )
            gwaits = []
            for b in range(NBUF):
                iwaits[b].wait()
                ws = []
                for j in range(K):
                    ws.append(
                        pltpu.async_copy(
                            table_hbm.at[idx_v.at[b].at[j]],
                            rows_v.at[b].at[j],
                            gsems[b],
                        )
                    )
                gwaits.append(ws)
            swaits = []
            for b in range(NBUF):
                for w in gwaits[b]:
                    w.wait()
                swaits.append(
                    pltpu.async_copy(
                        rows_v.at[b], out_w.at[pl.ds((c0 + b) * K, K)], ssems[b]
                    )
                )
            for w in swaits:
                w.wait()
            return carry

        lax.fori_loop(0, n_outer, outer_body, 0)

    return k


def kernel(tokens, table):
    B0, S = tokens.shape
    V, D = table.shape
    B = B0 * S
    assert B % _G == 0
    n_groups = B // _G
    info = plsc.get_sparse_core_info()
    NW = info.num_cores * info.num_subcores
    tokens3d = tokens.reshape(NW, n_groups // NW, _G)
    out = _make_gather(n_groups, V, D, K=5, NBUF=2)(tokens3d, table)
    return out.reshape(B0, S, D)


# speedup vs baseline: 1.0009x; 1.0009x over previous
import functools

import jax
import jax.numpy as jnp
from jax import lax
from jax.experimental import pallas as pl
from jax.experimental.pallas import tpu as pltpu
from jax.experimental.pallas import tpu_sc as plsc

_G = 128


def _make_gather(n_groups, V, D, K, NBUF):
    info = plsc.get_sparse_core_info()
    NC, NS = info.num_cores, info.num_subcores
    NW = NC * NS
    assert n_groups % NW == 0
    g_per_w = n_groups // NW
    assert g_per_w % (K * NBUF) == 0
    n_outer = g_per_w // (K * NBUF)
    mesh = plsc.VectorSubcoreMesh(core_axis_name="c", subcore_axis_name="s")

    sem_types = [pltpu.SemaphoreType.DMA] * (3 * NBUF)

    @functools.partial(
        pl.kernel,
        mesh=mesh,
        out_type=jax.ShapeDtypeStruct((NW, g_per_w, _G, D), jnp.float32),
        scratch_types=[
            pltpu.VMEM((NBUF, K, _G), jnp.int32),
            pltpu.VMEM((NBUF, K, _G, D), jnp.float32),
        ]
        + sem_types,
        compiler_params=pltpu.CompilerParams(use_tc_tiling_on_sc=False),
    )
    def k(tok_hbm, table_hbm, out_hbm, idx_v, rows_v, *sems):
        isems = sems[:NBUF]
        gsems = sems[NBUF : 2 * NBUF]
        ssems = sems[2 * NBUF :]
        wid = lax.axis_index("s") * NC + lax.axis_index("c")
        tok_w = tok_hbm.at[wid]
        out_w = out_hbm.at[wid]

        def outer_body(ci, carry):
            c0 = ci * NBUF
            iwaits = []
            for b in range(NBUF):
                iwaits.append(
                    pltpu.async_copy(
                        tok_w.at[pl.ds((c0 + b) * K, K)], idx_v.at[b], isems[b]
                    )
                )
            gwaits = []
            for b in range(NBUF):
                iwaits[b].wait()
                ws = []
                for j in range(K):
                    ws.append(
                        pltpu.async_copy(
                            table_hbm.at[idx_v.at[b].at[j]],
                            rows_v.at[b].at[j],
                            gsems[b],
                        )
                    )
                gwaits.append(ws)
            swaits = []
            for b in range(NBUF):
                for w in gwaits[b]:
                    w.wait()
                swaits.append(
                    pltpu.async_copy(
                        rows_v.at[b], out_w.at[pl.ds((c0 + b) * K, K)], ssems[b]
                    )
                )
            for w in swaits:
                w.wait()
            return carry

        lax.fori_loop(0, n_outer, outer_body, 0)

    return k


def kernel(tokens, table):
    B0, S = tokens.shape
    V, D = table.shape
    B = B0 * S
    assert B % _G == 0
    n_groups = B // _G
    info = plsc.get_sparse_core_info()
    NW = info.num_cores * info.num_subcores
    tokens3d = tokens.reshape(NW, n_groups // NW, _G)
    out = _make_gather(n_groups, V, D, K=5, NBUF=4)(tokens3d, table)
    return out.reshape(B0, S, D)


# speedup vs baseline: 1.0036x; 1.0027x over previous
import functools

import jax
import jax.numpy as jnp
from jax import lax
from jax.experimental import pallas as pl
from jax.experimental.pallas import tpu as pltpu
from jax.experimental.pallas import tpu_sc as plsc

_G = 128


def _make_gather(n_groups, V, D, K, NBUF):
    info = plsc.get_sparse_core_info()
    NC, NS = info.num_cores, info.num_subcores
    NW = NC * NS
    assert n_groups % NW == 0
    g_per_w = n_groups // NW
    assert g_per_w % (K * NBUF) == 0
    n_outer = g_per_w // (K * NBUF)
    mesh = plsc.VectorSubcoreMesh(core_axis_name="c", subcore_axis_name="s")

    sem_types = [pltpu.SemaphoreType.DMA] * (3 * NBUF)

    @functools.partial(
        pl.kernel,
        mesh=mesh,
        out_type=jax.ShapeDtypeStruct((NW, g_per_w, _G, D), jnp.float32),
        scratch_types=[
            pltpu.VMEM((NBUF, K, _G), jnp.int32),
            pltpu.VMEM((NBUF, K, _G, D), jnp.float32),
        ]
        + sem_types,
        compiler_params=pltpu.CompilerParams(use_tc_tiling_on_sc=False),
    )
    def k(tok_hbm, table_hbm, out_hbm, idx_v, rows_v, *sems):
        isems = sems[:NBUF]
        gsems = sems[NBUF : 2 * NBUF]
        ssems = sems[2 * NBUF :]
        wid = lax.axis_index("s") * NC + lax.axis_index("c")
        tok_w = tok_hbm.at[wid]
        out_w = out_hbm.at[wid]

        def wait_store(c, b):
            pltpu.make_async_copy(
                rows_v.at[b], out_w.at[pl.ds(c * K, K)], ssems[b]
            ).wait()

        def emit_iter(c0, first):
            iwaits = []
            for b in range(NBUF):
                iwaits.append(
                    pltpu.async_copy(
                        tok_w.at[pl.ds((c0 + b) * K, K)], idx_v.at[b], isems[b]
                    )
                )
            gwaits = []
            for b in range(NBUF):
                if not first:
                    wait_store(c0 + b - NBUF, b)
                iwaits[b].wait()
                ws = []
                for j in range(K):
                    ws.append(
                        pltpu.async_copy(
                            table_hbm.at[idx_v.at[b].at[j]],
                            rows_v.at[b].at[j],
                            gsems[b],
                        )
                    )
                gwaits.append(ws)
            for b in range(NBUF):
                for w in gwaits[b]:
                    w.wait()
                pltpu.async_copy(
                    rows_v.at[b], out_w.at[pl.ds((c0 + b) * K, K)], ssems[b]
                )

        emit_iter(0, True)

        def outer_body(ci, carry):
            emit_iter(ci * NBUF, False)
            return carry

        lax.fori_loop(1, n_outer, outer_body, 0)

        for b in range(NBUF):
            wait_store((n_outer - 1) * NBUF + b, b)

    return k


def kernel(tokens, table):
    B0, S = tokens.shape
    V, D = table.shape
    B = B0 * S
    assert B % _G == 0
    n_groups = B // _G
    info = plsc.get_sparse_core_info()
    NW = info.num_cores * info.num_subcores
    tokens3d = tokens.reshape(NW, n_groups // NW, _G)
    out = _make_gather(n_groups, V, D, K=5, NBUF=4)(tokens3d, table)
    return out.reshape(B0, S, D)
